# baseline (speedup 1.0000x reference)
import jax
import jax.numpy as jnp
from jax.experimental import pallas as pl
from jax.experimental.pallas import tpu as pltpu

_N = 65536
_D = 32
_K = 1024
_B = 64
_T = _N // _B


def _vq_kernel(x_ref, csq_ref, cb2_ref, cbsplit_ref, out_ref, loss_ref,
               perp_ref, counts_scr, err_scr):
    step = pl.program_id(0)

    @pl.when(step == 0)
    def _init():
        counts_scr[...] = jnp.zeros_like(counts_scr)
        err_scr[0, 0] = 0.0

    xt = jnp.concatenate([x_ref[0], x_ref[1], x_ref[2], x_ref[3]], axis=1)
    xsq = jnp.sum(xt * xt, axis=0, keepdims=True)
    csq = csq_ref[...]
    xc2 = jax.lax.dot_general(
        cb2_ref[...], xt, (((1,), (0,)), ((), ())),
        preferred_element_type=jnp.float32)
    dist = xsq - xc2 + csq
    m = jnp.min(dist, axis=0, keepdims=True)
    iota = jax.lax.broadcasted_iota(jnp.int32, (_K, 4 * _T), 0)
    idx = jnp.min(jnp.where(dist == m, iota, _K), axis=0, keepdims=True)
    oh = jnp.where(iota == idx, 1.0, 0.0)
    parts = jax.lax.dot_general(
        cbsplit_ref[...], oh.astype(jnp.bfloat16),
        (((1,), (0,)), ((), ())),
        preferred_element_type=jnp.float32)
    codes = parts[:_D] + parts[128:128 + _D]
    d = codes - xt
    res = xt + d
    out_ref[0] = res[:, :_T]
    out_ref[1] = res[:, _T:2 * _T]
    out_ref[2] = res[:, 2 * _T:3 * _T]
    out_ref[3] = res[:, 3 * _T:]
    err_scr[0, 0] += jnp.sum(d * d)
    counts_scr[...] += jnp.sum(oh, axis=1, keepdims=True)

    @pl.when(step == _B // 4 - 1)
    def _fin():
        p = counts_scr[...] * (1.0 / _N)
        ent = -jnp.sum(p * jnp.log(p + 1e-10))
        loss_ref[0, 0] = err_scr[0, 0] * (1.25 / (_N * _D))
        perp_ref[0, 0] = jnp.exp(ent)


def kernel(inputs, codebook):
    xt = jnp.transpose(inputs, (0, 2, 1))
    cb2 = codebook + codebook
    csq = jnp.sum(codebook ** 2, axis=-1, keepdims=True)
    hi32 = codebook.astype(jnp.bfloat16).astype(jnp.float32)
    mid = (codebook - hi32).astype(jnp.bfloat16)
    zpad = jnp.zeros((128 - _D, _K), jnp.bfloat16)
    cbsplit = jnp.concatenate(
        [hi32.astype(jnp.bfloat16).T, zpad, mid.T, zpad], axis=0)
    quantize_t, loss, perp = pl.pallas_call(
        _vq_kernel,
        grid=(_B // 4,),
        in_specs=[
            pl.BlockSpec((4, _D, _T), lambda i: (i, 0, 0)),
            pl.BlockSpec((_K, 1), lambda i: (0, 0)),
            pl.BlockSpec((_K, _D), lambda i: (0, 0)),
            pl.BlockSpec((256, _K), lambda i: (0, 0)),
        ],
        out_specs=[
            pl.BlockSpec((4, _D, _T), lambda i: (i, 0, 0)),
            pl.BlockSpec(memory_space=pltpu.SMEM),
            pl.BlockSpec(memory_space=pltpu.SMEM),
        ],
        out_shape=[
            jax.ShapeDtypeStruct((_B, _D, _T), jnp.float32),
            jax.ShapeDtypeStruct((1, 1), jnp.float32),
            jax.ShapeDtypeStruct((1, 1), jnp.float32),
        ],
        scratch_shapes=[
            pltpu.VMEM((_K, 1), jnp.float32),
            pltpu.SMEM((1, 1), jnp.float32),
        ],
    )(xt, csq, cb2, cbsplit)
    return (jnp.transpose(quantize_t, (0, 2, 1)), loss[0, 0], perp[0, 0])


# speedup vs baseline: 1.0006x; 1.0006x over previous
import jax
import jax.numpy as jnp
from jax.experimental import pallas as pl
from jax.experimental.pallas import tpu as pltpu

_N = 65536
_D = 32
_K = 1024
_B = 64
_T = _N // _B


def _vq_kernel(x_ref, csq_ref, cb2_ref, cbsplit_ref, out_ref, loss_ref,
               perp_ref, counts_scr, err_scr):
    step = pl.program_id(0)

    @pl.when(step == 0)
    def _init():
        counts_scr[...] = jnp.zeros_like(counts_scr)
        err_scr[0, 0] = 0.0

    xt = jnp.concatenate([x_ref[0], x_ref[1]], axis=1)
    xsq = jnp.sum(xt * xt, axis=0, keepdims=True)
    csq = csq_ref[...]
    xc2 = jax.lax.dot_general(
        cb2_ref[...], xt, (((1,), (0,)), ((), ())),
        preferred_element_type=jnp.float32)
    dist = xsq - xc2 + csq
    m = jnp.min(dist, axis=0, keepdims=True)
    iota = jax.lax.broadcasted_iota(jnp.int32, (_K, 2 * _T), 0)
    idx = jnp.min(jnp.where(dist == m, iota, _K), axis=0, keepdims=True)
    oh = jnp.where(iota == idx, 1.0, 0.0)
    parts = jax.lax.dot_general(
        cbsplit_ref[...], oh.astype(jnp.bfloat16),
        (((1,), (0,)), ((), ())),
        preferred_element_type=jnp.float32)
    codes = parts[:_D] + parts[128:128 + _D]
    d = codes - xt
    res = xt + d
    out_ref[0] = res[:, :_T]
    out_ref[1] = res[:, _T:]
    err_scr[0, 0] += jnp.sum(d * d)
    counts_scr[...] += jnp.sum(oh, axis=1, keepdims=True)

    @pl.when(step == _B // 2 - 1)
    def _fin():
        p = counts_scr[...] * (1.0 / _N)
        ent = -jnp.sum(p * jnp.log(p + 1e-10))
        loss_ref[0, 0] = err_scr[0, 0] * (1.25 / (_N * _D))
        perp_ref[0, 0] = jnp.exp(ent)


def kernel(inputs, codebook):
    xt = jnp.transpose(inputs, (0, 2, 1))
    cb2 = codebook + codebook
    csq = jnp.sum(codebook ** 2, axis=-1, keepdims=True)
    hi32 = codebook.astype(jnp.bfloat16).astype(jnp.float32)
    mid = (codebook - hi32).astype(jnp.bfloat16)
    zpad = jnp.zeros((128 - _D, _K), jnp.bfloat16)
    cbsplit = jnp.concatenate(
        [hi32.astype(jnp.bfloat16).T, zpad, mid.T, zpad], axis=0)
    quantize_t, loss, perp = pl.pallas_call(
        _vq_kernel,
        grid=(_B // 2,),
        in_specs=[
            pl.BlockSpec((2, _D, _T), lambda i: (i, 0, 0)),
            pl.BlockSpec((_K, 1), lambda i: (0, 0)),
            pl.BlockSpec((_K, _D), lambda i: (0, 0)),
            pl.BlockSpec((256, _K), lambda i: (0, 0)),
        ],
        out_specs=[
            pl.BlockSpec((2, _D, _T), lambda i: (i, 0, 0)),
            pl.BlockSpec(memory_space=pltpu.SMEM),
            pl.BlockSpec(memory_space=pltpu.SMEM),
        ],
        out_shape=[
            jax.ShapeDtypeStruct((_B, _D, _T), jnp.float32),
            jax.ShapeDtypeStruct((1, 1), jnp.float32),
            jax.ShapeDtypeStruct((1, 1), jnp.float32),
        ],
        scratch_shapes=[
            pltpu.VMEM((_K, 1), jnp.float32),
            pltpu.SMEM((1, 1), jnp.float32),
        ],
    )(xt, csq, cb2, cbsplit)
    return (jnp.transpose(quantize_t, (0, 2, 1)), loss[0, 0], perp[0, 0])
